# baseline (speedup 1.0000x reference)
import functools

import jax
import jax.numpy as jnp
from jax import lax
from jax.experimental import pallas as pl
from jax.experimental.pallas import tpu as pltpu
from jax.experimental.pallas import tpu_sc as plsc


_KNN_T = {}


def _extract_direct(dist, k):
    lane = lax.broadcasted_iota(jnp.int32, dist.shape, 1)
    big = jnp.int32(0x7FFFFFFF)
    cols = []
    for j in range(k):
        m = jnp.min(dist, axis=1, keepdims=True)
        hit = dist == m
        idxj = jnp.min(jnp.where(hit, lane, big), axis=1, keepdims=True)
        cols.append(idxj)
        if j + 1 < k:
            dist = jnp.where(lane == idxj, jnp.float32(jnp.inf), dist)
    return jnp.concatenate(cols, axis=1)


def _extract_chunked(dist, k, T, nc=16):
    R, N = dist.shape
    W = N // nc
    d2 = dist.reshape(R * nc, W)
    lane_l = lax.broadcasted_iota(jnp.int32, (R * nc, W), 1)
    crow = lax.broadcasted_iota(jnp.int32, (R * nc, 1), 0)
    lane_g = lane_l + (crow % nc) * W
    big = jnp.int32(0x7FFFFFFF)
    inf = jnp.float32(jnp.inf)
    vs, ls = [], []
    for t in range(T):
        m = jnp.min(d2, axis=1, keepdims=True)
        hit = d2 == m
        lsel = jnp.min(jnp.where(hit, lane_g, big), axis=1, keepdims=True)
        vs.append(m)
        ls.append(lsel)
        d2 = jnp.where(lane_g == lsel, inf, d2)
    tau = jnp.min(jnp.min(d2, axis=1, keepdims=True).reshape(R, nc),
                  axis=1, keepdims=True)
    V = jnp.concatenate(vs, axis=1).reshape(R, nc * T)
    L = jnp.concatenate(ls, axis=1).reshape(R, nc * T)
    cols = []
    for j in range(k):
        mv = jnp.min(V, axis=1, keepdims=True)
        hitv = V == mv
        lj = jnp.min(jnp.where(hitv, L, big), axis=1, keepdims=True)
        cols.append(lj)
        if j + 1 < k:
            V = jnp.where(hitv & (L == lj), inf, V)
    idx = jnp.concatenate(cols, axis=1)
    valid = jnp.all(mv < tau)
    return idx, valid


def _knn_body(xt_ref, x_ref, out_ref, *, k, n):
    b = pl.program_id(0)
    xtb = xt_ref[0]
    xb = x_ref[0]
    inner = lax.dot_general(xtb, xb, (((1,), (0,)), ((), ())),
                            preferred_element_type=jnp.float32)
    xx_row = jnp.sum(xb * xb, axis=0, keepdims=True)
    xx_col = jnp.sum(xtb * xtb, axis=1, keepdims=True)
    dist = (xx_col - 2.0 * inner) + xx_row
    base = b * n
    T = _KNN_T.get(k)
    if T is None:
        out = _extract_direct(dist, k)
    else:
        idx_c, valid = _extract_chunked(dist, k, T)
        out = lax.cond(valid,
                       lambda d: idx_c,
                       lambda d: _extract_direct(d, k),
                       dist)
    out_ref[0] = out + base


def _knn(xr, k):
    B, N, C = xr.shape
    x = xr.transpose(0, 2, 1)
    R = 1024
    return pl.pallas_call(
        functools.partial(_knn_body, k=k, n=N),
        grid=(B, N // R),
        in_specs=[pl.BlockSpec((1, R, C), lambda b, r: (b, r, 0)),
                  pl.BlockSpec((1, C, N), lambda b, r: (b, 0, 0))],
        out_specs=pl.BlockSpec((1, R, k), lambda b, r: (b, r, 0)),
        out_shape=jax.ShapeDtypeStruct((B, N, k), jnp.int32),
    )(xr, x)


def _mm_body(x_ref, w_ref, b_ref, o_ref, *, act):
    h = jnp.dot(x_ref[...], w_ref[...],
                preferred_element_type=jnp.float32) + b_ref[...]
    o_ref[...] = jnp.maximum(h, 0.0) if act else h


def _mm(x, w, b, act, R=1024):
    M, Cin = x.shape
    Cout = w.shape[1]
    return pl.pallas_call(
        functools.partial(_mm_body, act=act),
        grid=(M // R,),
        in_specs=[pl.BlockSpec((R, Cin), lambda i: (i, 0)),
                  pl.BlockSpec((Cin, Cout), lambda i: (0, 0)),
                  pl.BlockSpec((1, Cout), lambda i: (0, 0))],
        out_specs=pl.BlockSpec((R, Cout), lambda i: (i, 0)),
        out_shape=jax.ShapeDtypeStruct((M, Cout), jnp.float32),
    )(x, w, b.reshape(1, -1))


def _edgek_body(g_ref, x_ref, w1_ref, b1_ref, w2_ref, b2_ref, o_ref,
                *, k, agg, resid):
    R, CP = x_ref.shape
    xc = x_ref[...]
    g = g_ref[...].reshape(R, k, CP)
    xb = xc.reshape(R, 1, CP)
    diff = (g - xb).reshape(R * k, CP)
    ctr = jnp.broadcast_to(xb, (R, k, CP)).reshape(R * k, CP)
    f = jnp.concatenate([diff, ctr], axis=1)
    h = jnp.maximum(jnp.dot(f, w1_ref[...],
                            preferred_element_type=jnp.float32)
                    + b1_ref[...], 0.0)
    if w2_ref is not None:
        h = jnp.maximum(jnp.dot(h, w2_ref[...],
                                preferred_element_type=jnp.float32)
                        + b2_ref[...], 0.0)
    C1 = h.shape[1]
    hr = h.reshape(R, k, C1)
    o = jnp.max(hr, axis=1) if agg == 'max' else jnp.sum(hr, axis=1)
    if resid:
        o = o + xc
    o_ref[...] = o


def _edgek(g, x, w1, b1, w2, b2, k, agg='max', resid=False, R=512):
    M, CP = x.shape
    C1 = w1.shape[1]
    C2 = C1 if w2 is None else w2.shape[1]
    specs = [pl.BlockSpec((R * k, CP), lambda i: (i, 0)),
             pl.BlockSpec((R, CP), lambda i: (i, 0)),
             pl.BlockSpec((2 * CP, C1), lambda i: (0, 0)),
             pl.BlockSpec((1, C1), lambda i: (0, 0))]
    args = [g, x, w1, b1.reshape(1, -1)]
    if w2 is not None:
        specs += [pl.BlockSpec((C1, C2), lambda i: (0, 0)),
                  pl.BlockSpec((1, C2), lambda i: (0, 0))]
        args += [w2, b2.reshape(1, -1)]
        body = functools.partial(_edgek_body, k=k, agg=agg, resid=resid)
    else:
        body = functools.partial(
            lambda g_r, x_r, w1_r, b1_r, o_r, k, agg, resid: _edgek_body(
                g_r, x_r, w1_r, b1_r, None, None, o_r,
                k=k, agg=agg, resid=resid),
            k=k, agg=agg, resid=resid)
    return pl.pallas_call(
        body,
        grid=(M // R,),
        in_specs=specs,
        out_specs=pl.BlockSpec((R, C2), lambda i: (i, 0)),
        out_shape=jax.ShapeDtypeStruct((M, C2), jnp.float32),
    )(*args)


def _sc_gather(table, idx):
    M = idx.shape[0]
    T, D = table.shape
    NW = 32
    per_w = M // NW
    CH = 128
    nch = per_w // CH
    mesh = plsc.VectorSubcoreMesh(core_axis_name="c", subcore_axis_name="s")

    @functools.partial(
        pl.kernel,
        out_type=jax.ShapeDtypeStruct((M, D), jnp.float32),
        mesh=mesh,
        scratch_types=[pltpu.VMEM((2, CH), jnp.int32),
                       pltpu.VMEM((2, CH, D), jnp.float32),
                       pltpu.SemaphoreType.DMA],
    )
    def kfn(table_hbm, idx_hbm, out_hbm, idx_v, rows_v, gsem):
        wid = lax.axis_index("s") * 2 + lax.axis_index("c")
        base = wid * per_w
        pltpu.sync_copy(idx_hbm.at[pl.ds(base, CH)], idx_v.at[0])
        pltpu.async_copy(table_hbm.at[idx_v.at[0]], rows_v.at[0], gsem)

        def body(i, carry):
            cur = lax.rem(i, 2)
            nxt = 1 - cur
            pltpu.make_async_copy(table_hbm.at[idx_v.at[cur]],
                                  rows_v.at[cur], gsem).wait()

            @pl.when(i + 1 < nch)
            def _prefetch():
                off = base + (i + 1) * CH
                pltpu.sync_copy(idx_hbm.at[pl.ds(off, CH)], idx_v.at[nxt])
                pltpu.async_copy(table_hbm.at[idx_v.at[nxt]],
                                 rows_v.at[nxt], gsem)

            pltpu.sync_copy(rows_v.at[cur],
                            out_hbm.at[pl.ds(base + i * CH, CH)])
            return carry

        lax.fori_loop(0, nch, body, 0)

    return kfn(table, idx)


def _mlp3_body(x_ref, w1_ref, b1_ref, w2_ref, b2_ref, w3_ref, b3_ref, o_ref):
    h = jnp.maximum(jnp.dot(x_ref[...], w1_ref[...],
                            preferred_element_type=jnp.float32)
                    + b1_ref[...], 0.0)
    h = jnp.maximum(jnp.dot(h, w2_ref[...],
                            preferred_element_type=jnp.float32)
                    + b2_ref[...], 0.0)
    o_ref[...] = (jnp.dot(h, w3_ref[...],
                          preferred_element_type=jnp.float32) + b3_ref[...])


def _mlp3(x, w1, b1, w2, b2, w3, b3, R=1024):
    M, Cin = x.shape
    c1, c2, c3 = w1.shape[1], w2.shape[1], w3.shape[1]
    return pl.pallas_call(
        _mlp3_body,
        grid=(M // R,),
        in_specs=[pl.BlockSpec((R, Cin), lambda i: (i, 0)),
                  pl.BlockSpec((Cin, c1), lambda i: (0, 0)),
                  pl.BlockSpec((1, c1), lambda i: (0, 0)),
                  pl.BlockSpec((c1, c2), lambda i: (0, 0)),
                  pl.BlockSpec((1, c2), lambda i: (0, 0)),
                  pl.BlockSpec((c2, c3), lambda i: (0, 0)),
                  pl.BlockSpec((1, c3), lambda i: (0, 0))],
        out_specs=pl.BlockSpec((R, c3), lambda i: (i, 0)),
        out_shape=jax.ShapeDtypeStruct((M, c3), jnp.float32),
    )(x, w1, b1.reshape(1, -1), w2, b2.reshape(1, -1), w3, b3.reshape(1, -1))


def _final_body(gm_ref, edge_ref, pos_ref, o_ref, mask_ref):
    g = gm_ref[...]
    mask_ref[...] = jnp.maximum(g, 0.0)
    keep = (g > 0.01).astype(jnp.float32)
    o_ref[...] = pos_ref[...] + edge_ref[...] * keep


def _final(gmask, edge24, pos24, R=1024):
    M = gmask.shape[0]
    return pl.pallas_call(
        _final_body,
        grid=(M // R,),
        in_specs=[pl.BlockSpec((R, 1), lambda i: (i, 0)),
                  pl.BlockSpec((R, 24), lambda i: (i, 0)),
                  pl.BlockSpec((R, 24), lambda i: (i, 0))],
        out_specs=[pl.BlockSpec((R, 24), lambda i: (i, 0)),
                   pl.BlockSpec((R, 1), lambda i: (i, 0))],
        out_shape=[jax.ShapeDtypeStruct((M, 24), jnp.float32),
                   jax.ShapeDtypeStruct((M, 1), jnp.float32)],
    )(gmask, edge24, pos24)


def _w1cat(w1, c, cp):
    C1 = w1.shape[0]
    wa = w1[:, :c].T
    wb = w1[:, c:].T
    if cp == c:
        return jnp.concatenate([wa, wb], axis=0)
    z = jnp.zeros((cp - c, C1), jnp.float32)
    return jnp.concatenate([wa, z, wb, z], axis=0)


def _pad128(wT, b):
    co = wT.shape[1]
    return jnp.pad(wT, ((0, 0), (0, 128 - co))), jnp.pad(b, (0, 128 - co))


def kernel(feature, pos, params):
    p = params
    B, N, _ = feature.shape
    BN = B * N

    f0 = feature.reshape(BN, 3)
    x128 = jnp.pad(f0, ((0, 0), (0, 125)))
    idx = _knn(x128.reshape(B, N, 128), 20)
    g = _sc_gather(x128, idx.reshape(-1))
    x1 = _edgek(g, x128, _w1cat(p['fe0_w1'], 3, 128), p['fe0_b1'],
                p['fe0_w2'].T, p['fe0_b2'], k=20)

    idx = _knn(x1.reshape(B, N, 128), 20)
    g = _sc_gather(x1, idx.reshape(-1))
    f1 = _edgek(g, x1, _w1cat(p['fe1_w'], 128, 128), p['fe1_b'],
                None, None, k=20, resid=True)

    idx = _knn(f1.reshape(B, N, 128), 20)
    g = _sc_gather(f1, idx.reshape(-1))
    f2 = _edgek(g, f1, _w1cat(p['fe2_w'], 128, 128), p['fe2_b'],
                None, None, k=20, resid=True)

    enc = jnp.concatenate([f1, f2], axis=1)

    w0, b0 = _pad128(p['up0_w'].T, p['up0_b'])
    h = _mm(enc, w0, b0, act=True)
    w0, b0 = _pad128(p['fb0_w'].T, p['fb0_b'])
    gb = _mm(enc, w0, b0, act=True)

    idxu = _knn(h.reshape(B, N, 128), 12)
    gu = _sc_gather(h, idxu.reshape(-1))
    idxf = _knn(gb.reshape(B, N, 128), 12)
    gf = _sc_gather(gb, idxf.reshape(-1))
    h = _edgek(gu, h, _w1cat(p['up1_w1'], 64, 128), p['up1_b1'],
               p['up1_w2'].T, p['up1_b2'], k=12)
    gb = _edgek(gf, gb, _w1cat(p['fb1_w1'], 64, 128), p['fb1_b1'],
                p['fb1_w2'].T, p['fb1_b2'], k=12)

    w0, b0 = _pad128(p['up2_w'].T, p['up2_b'])
    h = _mm(h, w0, b0, act=True)
    w0, b0 = _pad128(p['fb2_w'].T, p['fb2_b'])
    gb = _mm(gb, w0, b0, act=True)

    idxu = _knn(h.reshape(B, N, 128), 4)
    gu = _sc_gather(h, idxu.reshape(-1))
    idxf = _knn(gb.reshape(B, N, 128), 8)
    gf = _sc_gather(gb, idxf.reshape(-1))
    h = _edgek(gu, h, _w1cat(p['up3_w1'], 64, 128), p['up3_b1'],
               p['up3_w2'].T, p['up3_b2'], k=4)
    gb = _edgek(gf, gb, _w1cat(p['fb3_w'], 64, 128), p['fb3_b'],
                None, None, k=8, agg='sum')

    edge24 = _mlp3(h, p['updec_w1'].T, p['updec_b1'],
                   p['updec_w2'].T, p['updec_b2'],
                   p['updec_w3'].T, p['updec_b3'])
    gmask = _mlp3(gb, p['fbdec_w1'].T, p['fbdec_b1'],
                  p['fbdec_w2'].T, p['fbdec_b2'],
                  p['fbdec_w3'].T, p['fbdec_b3'])

    pos24 = jnp.concatenate([pos] * 8, axis=2).reshape(BN, 24)
    out24, mask = _final(gmask, edge24, pos24)
    return out24.reshape(B, N * 8, 3), mask.reshape(B, N, 1)


# speedup vs baseline: 1.1312x; 1.1312x over previous
import functools

import jax
import jax.numpy as jnp
from jax import lax
from jax.experimental import pallas as pl
from jax.experimental.pallas import tpu as pltpu
from jax.experimental.pallas import tpu_sc as plsc


_KNN_T = {}


def _extract_direct(dist, k):
    lane = lax.broadcasted_iota(jnp.int32, dist.shape, 1)
    big = jnp.int32(0x7FFFFFFF)
    cols = []
    for j in range(k):
        m = jnp.min(dist, axis=1, keepdims=True)
        hit = dist == m
        idxj = jnp.min(jnp.where(hit, lane, big), axis=1, keepdims=True)
        cols.append(idxj)
        if j + 1 < k:
            dist = jnp.where(lane == idxj, jnp.float32(jnp.inf), dist)
    return jnp.concatenate(cols, axis=1)


def _extract_chunked(dist, k, T, nc=16):
    R, N = dist.shape
    W = N // nc
    d2 = dist.reshape(R * nc, W)
    lane_l = lax.broadcasted_iota(jnp.int32, (R * nc, W), 1)
    crow = lax.broadcasted_iota(jnp.int32, (R * nc, 1), 0)
    lane_g = lane_l + (crow % nc) * W
    big = jnp.int32(0x7FFFFFFF)
    inf = jnp.float32(jnp.inf)
    vs, ls = [], []
    for t in range(T):
        m = jnp.min(d2, axis=1, keepdims=True)
        hit = d2 == m
        lsel = jnp.min(jnp.where(hit, lane_g, big), axis=1, keepdims=True)
        vs.append(m)
        ls.append(lsel)
        d2 = jnp.where(lane_g == lsel, inf, d2)
    tau = jnp.min(jnp.min(d2, axis=1, keepdims=True).reshape(R, nc),
                  axis=1, keepdims=True)
    V = jnp.concatenate(vs, axis=1).reshape(R, nc * T)
    L = jnp.concatenate(ls, axis=1).reshape(R, nc * T)
    cols = []
    for j in range(k):
        mv = jnp.min(V, axis=1, keepdims=True)
        hitv = V == mv
        lj = jnp.min(jnp.where(hitv, L, big), axis=1, keepdims=True)
        cols.append(lj)
        if j + 1 < k:
            V = jnp.where(hitv & (L == lj), inf, V)
    idx = jnp.concatenate(cols, axis=1)
    valid = jnp.all(mv < tau)
    return idx, valid


def _knn_body(xt_ref, x_ref, out_ref, *, k, n):
    b = pl.program_id(0)
    xtb = xt_ref[0]
    xb = x_ref[0]
    inner = lax.dot_general(xtb, xb, (((1,), (0,)), ((), ())),
                            preferred_element_type=jnp.float32)
    xx_row = jnp.sum(xb * xb, axis=0, keepdims=True)
    xx_col = jnp.sum(xtb * xtb, axis=1, keepdims=True)
    dist = (xx_col - 2.0 * inner) + xx_row
    base = b * n
    T = _KNN_T.get(k)
    if T is None:
        out = _extract_direct(dist, k)
    else:
        idx_c, valid = _extract_chunked(dist, k, T)
        out = lax.cond(valid,
                       lambda d: idx_c,
                       lambda d: _extract_direct(d, k),
                       dist)
    out_ref[0] = out + base


def _knn(xr, k):
    B, N, C = xr.shape
    x = xr.transpose(0, 2, 1)
    R = 512
    return pl.pallas_call(
        functools.partial(_knn_body, k=k, n=N),
        grid=(B, N // R),
        in_specs=[pl.BlockSpec((1, R, C), lambda b, r: (b, r, 0)),
                  pl.BlockSpec((1, C, N), lambda b, r: (b, 0, 0))],
        out_specs=pl.BlockSpec((1, R, k), lambda b, r: (b, r, 0)),
        out_shape=jax.ShapeDtypeStruct((B, N, k), jnp.int32),
    )(xr, x)


def _mm_body(x_ref, w_ref, b_ref, o_ref, *, act):
    h = jnp.dot(x_ref[...], w_ref[...],
                preferred_element_type=jnp.float32) + b_ref[...]
    o_ref[...] = jnp.maximum(h, 0.0) if act else h


def _mm(x, w, b, act, R=1024):
    M, Cin = x.shape
    Cout = w.shape[1]
    return pl.pallas_call(
        functools.partial(_mm_body, act=act),
        grid=(M // R,),
        in_specs=[pl.BlockSpec((R, Cin), lambda i: (i, 0)),
                  pl.BlockSpec((Cin, Cout), lambda i: (0, 0)),
                  pl.BlockSpec((1, Cout), lambda i: (0, 0))],
        out_specs=pl.BlockSpec((R, Cout), lambda i: (i, 0)),
        out_shape=jax.ShapeDtypeStruct((M, Cout), jnp.float32),
    )(x, w, b.reshape(1, -1))


def _edgek_body(g_ref, x_ref, w1_ref, b1_ref, w2_ref, b2_ref, o_ref,
                *, k, agg, resid):
    R, CP = x_ref.shape
    xc = x_ref[...]
    g = g_ref[...].reshape(R, k, CP)
    xb = xc.reshape(R, 1, CP)
    diff = (g - xb).reshape(R * k, CP)
    ctr = jnp.broadcast_to(xb, (R, k, CP)).reshape(R * k, CP)
    f = jnp.concatenate([diff, ctr], axis=1)
    h = jnp.maximum(jnp.dot(f, w1_ref[...],
                            preferred_element_type=jnp.float32)
                    + b1_ref[...], 0.0)
    if w2_ref is not None:
        h = jnp.maximum(jnp.dot(h, w2_ref[...],
                                preferred_element_type=jnp.float32)
                        + b2_ref[...], 0.0)
    C1 = h.shape[1]
    hr = h.reshape(R, k, C1)
    o = jnp.max(hr, axis=1) if agg == 'max' else jnp.sum(hr, axis=1)
    if resid:
        o = o + xc
    o_ref[...] = o


def _edgek(g, x, w1, b1, w2, b2, k, agg='max', resid=False, R=512):
    M, CP = x.shape
    C1 = w1.shape[1]
    C2 = C1 if w2 is None else w2.shape[1]
    specs = [pl.BlockSpec((R * k, CP), lambda i: (i, 0)),
             pl.BlockSpec((R, CP), lambda i: (i, 0)),
             pl.BlockSpec((2 * CP, C1), lambda i: (0, 0)),
             pl.BlockSpec((1, C1), lambda i: (0, 0))]
    args = [g, x, w1, b1.reshape(1, -1)]
    if w2 is not None:
        specs += [pl.BlockSpec((C1, C2), lambda i: (0, 0)),
                  pl.BlockSpec((1, C2), lambda i: (0, 0))]
        args += [w2, b2.reshape(1, -1)]
        body = functools.partial(_edgek_body, k=k, agg=agg, resid=resid)
    else:
        body = functools.partial(
            lambda g_r, x_r, w1_r, b1_r, o_r, k, agg, resid: _edgek_body(
                g_r, x_r, w1_r, b1_r, None, None, o_r,
                k=k, agg=agg, resid=resid),
            k=k, agg=agg, resid=resid)
    return pl.pallas_call(
        body,
        grid=(M // R,),
        in_specs=specs,
        out_specs=pl.BlockSpec((R, C2), lambda i: (i, 0)),
        out_shape=jax.ShapeDtypeStruct((M, C2), jnp.float32),
    )(*args)


def _sc_gather(table, idx):
    M = idx.shape[0]
    T, D = table.shape
    NW = 32
    per_w = M // NW
    CH = 128
    nch = per_w // CH
    mesh = plsc.VectorSubcoreMesh(core_axis_name="c", subcore_axis_name="s")

    @functools.partial(
        pl.kernel,
        out_type=jax.ShapeDtypeStruct((M, D), jnp.float32),
        mesh=mesh,
        scratch_types=[pltpu.VMEM((2, CH), jnp.int32),
                       pltpu.VMEM((2, CH, D), jnp.float32),
                       pltpu.SemaphoreType.DMA],
    )
    def kfn(table_hbm, idx_hbm, out_hbm, idx_v, rows_v, gsem):
        wid = lax.axis_index("s") * 2 + lax.axis_index("c")
        base = wid * per_w
        pltpu.sync_copy(idx_hbm.at[pl.ds(base, CH)], idx_v.at[0])
        pltpu.async_copy(table_hbm.at[idx_v.at[0]], rows_v.at[0], gsem)

        def body(i, carry):
            cur = lax.rem(i, 2)
            nxt = 1 - cur
            pltpu.make_async_copy(table_hbm.at[idx_v.at[cur]],
                                  rows_v.at[cur], gsem).wait()

            @pl.when(i + 1 < nch)
            def _prefetch():
                off = base + (i + 1) * CH
                pltpu.sync_copy(idx_hbm.at[pl.ds(off, CH)], idx_v.at[nxt])
                pltpu.async_copy(table_hbm.at[idx_v.at[nxt]],
                                 rows_v.at[nxt], gsem)

            pltpu.sync_copy(rows_v.at[cur],
                            out_hbm.at[pl.ds(base + i * CH, CH)])
            return carry

        lax.fori_loop(0, nch, body, 0)

    return kfn(table, idx)


def _mlp3_body(x_ref, w1_ref, b1_ref, w2_ref, b2_ref, w3_ref, b3_ref, o_ref):
    h = jnp.maximum(jnp.dot(x_ref[...], w1_ref[...],
                            preferred_element_type=jnp.float32)
                    + b1_ref[...], 0.0)
    h = jnp.maximum(jnp.dot(h, w2_ref[...],
                            preferred_element_type=jnp.float32)
                    + b2_ref[...], 0.0)
    o_ref[...] = (jnp.dot(h, w3_ref[...],
                          preferred_element_type=jnp.float32) + b3_ref[...])


def _mlp3(x, w1, b1, w2, b2, w3, b3, R=1024):
    M, Cin = x.shape
    c1, c2, c3 = w1.shape[1], w2.shape[1], w3.shape[1]
    return pl.pallas_call(
        _mlp3_body,
        grid=(M // R,),
        in_specs=[pl.BlockSpec((R, Cin), lambda i: (i, 0)),
                  pl.BlockSpec((Cin, c1), lambda i: (0, 0)),
                  pl.BlockSpec((1, c1), lambda i: (0, 0)),
                  pl.BlockSpec((c1, c2), lambda i: (0, 0)),
                  pl.BlockSpec((1, c2), lambda i: (0, 0)),
                  pl.BlockSpec((c2, c3), lambda i: (0, 0)),
                  pl.BlockSpec((1, c3), lambda i: (0, 0))],
        out_specs=pl.BlockSpec((R, c3), lambda i: (i, 0)),
        out_shape=jax.ShapeDtypeStruct((M, c3), jnp.float32),
    )(x, w1, b1.reshape(1, -1), w2, b2.reshape(1, -1), w3, b3.reshape(1, -1))


def _final_body(gm_ref, edge_ref, pos_ref, o_ref, mask_ref):
    g = gm_ref[...]
    mask_ref[...] = jnp.maximum(g, 0.0)
    keep = (g > 0.01).astype(jnp.float32)
    o_ref[...] = pos_ref[...] + edge_ref[...] * keep


def _final(gmask, edge24, pos24, R=1024):
    M = gmask.shape[0]
    return pl.pallas_call(
        _final_body,
        grid=(M // R,),
        in_specs=[pl.BlockSpec((R, 1), lambda i: (i, 0)),
                  pl.BlockSpec((R, 24), lambda i: (i, 0)),
                  pl.BlockSpec((R, 24), lambda i: (i, 0))],
        out_specs=[pl.BlockSpec((R, 24), lambda i: (i, 0)),
                   pl.BlockSpec((R, 1), lambda i: (i, 0))],
        out_shape=[jax.ShapeDtypeStruct((M, 24), jnp.float32),
                   jax.ShapeDtypeStruct((M, 1), jnp.float32)],
    )(gmask, edge24, pos24)


def _w1cat(w1, c, cp):
    C1 = w1.shape[0]
    wa = w1[:, :c].T
    wb = w1[:, c:].T
    if cp == c:
        return jnp.concatenate([wa, wb], axis=0)
    z = jnp.zeros((cp - c, C1), jnp.float32)
    return jnp.concatenate([wa, z, wb, z], axis=0)


def _pad128(wT, b):
    co = wT.shape[1]
    return jnp.pad(wT, ((0, 0), (0, 128 - co))), jnp.pad(b, (0, 128 - co))


def kernel(feature, pos, params):
    p = params
    B, N, _ = feature.shape
    BN = B * N

    f0 = feature.reshape(BN, 3)
    x128 = jnp.pad(f0, ((0, 0), (0, 125)))
    idx = _knn(x128.reshape(B, N, 128), 20)
    g = _sc_gather(x128, idx.reshape(-1))
    x1 = _edgek(g, x128, _w1cat(p['fe0_w1'], 3, 128), p['fe0_b1'],
                p['fe0_w2'].T, p['fe0_b2'], k=20)

    idx = _knn(x1.reshape(B, N, 128), 20)
    g = _sc_gather(x1, idx.reshape(-1))
    f1 = _edgek(g, x1, _w1cat(p['fe1_w'], 128, 128), p['fe1_b'],
                None, None, k=20, resid=True)

    idx = _knn(f1.reshape(B, N, 128), 20)
    g = _sc_gather(f1, idx.reshape(-1))
    f2 = _edgek(g, f1, _w1cat(p['fe2_w'], 128, 128), p['fe2_b'],
                None, None, k=20, resid=True)

    enc = jnp.concatenate([f1, f2], axis=1)

    w0, b0 = _pad128(p['up0_w'].T, p['up0_b'])
    h = _mm(enc, w0, b0, act=True)
    w0, b0 = _pad128(p['fb0_w'].T, p['fb0_b'])
    gb = _mm(enc, w0, b0, act=True)

    idxu = _knn(h.reshape(B, N, 128), 12)
    gu = _sc_gather(h, idxu.reshape(-1))
    idxf = _knn(gb.reshape(B, N, 128), 12)
    gf = _sc_gather(gb, idxf.reshape(-1))
    h = _edgek(gu, h, _w1cat(p['up1_w1'], 64, 128), p['up1_b1'],
               p['up1_w2'].T, p['up1_b2'], k=12)
    gb = _edgek(gf, gb, _w1cat(p['fb1_w1'], 64, 128), p['fb1_b1'],
                p['fb1_w2'].T, p['fb1_b2'], k=12)

    w0, b0 = _pad128(p['up2_w'].T, p['up2_b'])
    h = _mm(h, w0, b0, act=True)
    w0, b0 = _pad128(p['fb2_w'].T, p['fb2_b'])
    gb = _mm(gb, w0, b0, act=True)

    idxu = _knn(h.reshape(B, N, 128), 4)
    gu = _sc_gather(h, idxu.reshape(-1))
    idxf = _knn(gb.reshape(B, N, 128), 8)
    gf = _sc_gather(gb, idxf.reshape(-1))
    h = _edgek(gu, h, _w1cat(p['up3_w1'], 64, 128), p['up3_b1'],
               p['up3_w2'].T, p['up3_b2'], k=4)
    gb = _edgek(gf, gb, _w1cat(p['fb3_w'], 64, 128), p['fb3_b'],
                None, None, k=8, agg='sum')

    edge24 = _mlp3(h, p['updec_w1'].T, p['updec_b1'],
                   p['updec_w2'].T, p['updec_b2'],
                   p['updec_w3'].T, p['updec_b3'])
    gmask = _mlp3(gb, p['fbdec_w1'].T, p['fbdec_b1'],
                  p['fbdec_w2'].T, p['fbdec_b2'],
                  p['fbdec_w3'].T, p['fbdec_b3'])

    pos24 = jnp.concatenate([pos] * 8, axis=2).reshape(BN, 24)
    out24, mask = _final(gmask, edge24, pos24)
    return out24.reshape(B, N * 8, 3), mask.reshape(B, N, 1)
